# Initial kernel scaffold; baseline (speedup 1.0000x reference)
import jax
import jax.numpy as jnp
from jax.experimental import pallas as pl


def kernel(x, embed_weight, proj_weight, proj_bias):
    raise NotImplementedError("write your pallas kernel here")


# speedup vs baseline: 5.5910x; 5.5910x over previous
import functools

import jax
import jax.numpy as jnp
from jax import lax
from jax.experimental import pallas as pl
from jax.experimental.pallas import tpu as pltpu
from jax.experimental.pallas import tpu_sc as plsc

_NC = 2
_NS = 16
_L = 16
_NW = _NC * _NS

_D = 8
_V = 16
_CHUNK = 4096


def _tiny_model_body(n_per_w, x_hbm, emb_hbm, w_hbm, b_hbm, out_hbm,
                     tab_v, idx_v, out_v):
    wid = lax.axis_index("s") * _NC + lax.axis_index("c")
    base = wid * n_per_w

    lanes = lax.iota(jnp.int32, _L)
    o_pat = lanes & 7
    half = lanes >> 3

    def with_weights(emb_v, w_v, b_v):
        pltpu.sync_copy(emb_hbm, emb_v)
        pltpu.sync_copy(w_hbm, w_v)
        pltpu.sync_copy(b_hbm, b_v)

        def build_tab(j, carry):
            k_pat = 2 * j + half
            acc = plsc.load_gather(b_v, [o_pat])
            for d in range(_D):
                ev = plsc.load_gather(emb_v, [k_pat * _D + d])
                wv = plsc.load_gather(w_v, [o_pat * _D + d])
                acc = acc + ev * wv
            tab_v[pl.ds(j * _L, _L)] = acc
            return carry

        lax.fori_loop(0, _V // 2, build_tab, 0)

    pl.run_scoped(
        with_weights,
        pltpu.VMEM((128,), jnp.float32),
        pltpu.VMEM((128,), jnp.float32),
        pltpu.VMEM((128,), jnp.float32),
    )

    n_chunks = n_per_w // _CHUNK
    n_groups = _CHUNK // _L

    def do_chunk(g, carry):
        start = base + g * _CHUNK
        pltpu.sync_copy(x_hbm.at[pl.ds(start, _CHUNK)], idx_v)

        def group(i, c):
            xv = idx_v[pl.ds(i * _L, _L)]
            xv8 = xv * _D
            n8 = (i * _L + lanes) * _D
            for o in range(_D):
                val = plsc.load_gather(tab_v, [xv8 + o])
                plsc.store_scatter(out_v, [n8 + o], val)
            return c

        lax.fori_loop(0, n_groups, group, 0)
        pltpu.sync_copy(out_v, out_hbm.at[pl.ds(start * _D, _CHUNK * _D)])
        return carry

    lax.fori_loop(0, n_chunks, do_chunk, 0)


def _make_sc_call(n_total):
    n_per_w = n_total // _NW
    mesh = plsc.VectorSubcoreMesh(core_axis_name="c", subcore_axis_name="s")
    return pl.kernel(
        functools.partial(_tiny_model_body, n_per_w),
        out_type=jax.ShapeDtypeStruct((n_total * _D,), jnp.float32),
        mesh=mesh,
        compiler_params=pltpu.CompilerParams(needs_layout_passes=False),
        scratch_types=[
            pltpu.VMEM((_V * _D,), jnp.float32),
            pltpu.VMEM((_CHUNK,), jnp.int32),
            pltpu.VMEM((_CHUNK * _D,), jnp.float32),
        ],
    )


def kernel(x, embed_weight, proj_weight, proj_bias):
    b, l = x.shape
    n_total = b * l
    x_flat = x.reshape(-1).astype(jnp.int32)
    out_flat = _make_sc_call(n_total)(
        x_flat,
        embed_weight.reshape(-1).astype(jnp.float32),
        jnp.pad(proj_weight.reshape(-1).astype(jnp.float32), (0, 128 - _D * _D)),
        jnp.pad(proj_bias.astype(jnp.float32), (0, 128 - _D)),
    )
    return out_flat.reshape(b, l, _D)


# speedup vs baseline: 6.3220x; 1.1307x over previous
import functools

import jax
import jax.numpy as jnp
from jax import lax
from jax.experimental import pallas as pl
from jax.experimental.pallas import tpu as pltpu
from jax.experimental.pallas import tpu_sc as plsc

_NC = 2
_NS = 16
_L = 16
_NW = _NC * _NS

_D = 8
_V = 16
_CHUNK = 4096


def _tiny_model_body(n_per_w, x_hbm, emb_hbm, w_hbm, b_hbm, out_hbm,
                     tab_v, idx_v0, idx_v1, out_v0, out_v1, sem_in, sem_out):
    idx_v = [idx_v0, idx_v1]
    out_v = [out_v0, out_v1]
    wid = lax.axis_index("s") * _NC + lax.axis_index("c")
    base = wid * n_per_w

    lanes = lax.iota(jnp.int32, _L)
    o_pat = lanes & 7
    half = lanes >> 3

    def with_weights(emb_v, w_v, b_v):
        pltpu.sync_copy(emb_hbm, emb_v)
        pltpu.sync_copy(w_hbm, w_v)
        pltpu.sync_copy(b_hbm, b_v)

        def build_tab(j, carry):
            k_pat = 2 * j + half
            acc = plsc.load_gather(b_v, [o_pat])
            for d in range(_D):
                ev = plsc.load_gather(emb_v, [k_pat * _D + d])
                wv = plsc.load_gather(w_v, [o_pat * _D + d])
                acc = acc + ev * wv
            tab_v[pl.ds(j * _L, _L)] = acc
            return carry

        lax.fori_loop(0, _V // 2, build_tab, 0)

    pl.run_scoped(
        with_weights,
        pltpu.VMEM((128,), jnp.float32),
        pltpu.VMEM((128,), jnp.float32),
        pltpu.VMEM((128,), jnp.float32),
    )

    n_chunks = n_per_w // _CHUNK
    n_groups = _CHUNK // _L
    lanes8 = lanes * _D

    in_cp = [None, None]
    out_cp = [None, None]

    def compute_chunk(bf):
        idx_b = idx_v[bf]
        out_b = out_v[bf]

        def group(i):
            xv = idx_b[pl.ds(i * _L, _L)]
            xv8 = xv * _D
            n8 = i * (_L * _D) + lanes8
            for o in range(_D):
                val = plsc.load_gather(tab_v, [xv8 + o])
                plsc.store_scatter(out_b, [n8 + o], val)

        plsc.parallel_loop(0, n_groups, 1, unroll=8)(group)

    in_cp[0] = pltpu.async_copy(
        x_hbm.at[pl.ds(base, _CHUNK)], idx_v[0], sem_in[0])
    for g in range(n_chunks):
        bf = g & 1
        in_cp[bf].wait()
        if g + 1 < n_chunks:
            in_cp[1 - bf] = pltpu.async_copy(
                x_hbm.at[pl.ds(base + (g + 1) * _CHUNK, _CHUNK)],
                idx_v[1 - bf], sem_in[1 - bf])
        if g >= 2:
            out_cp[bf].wait()
        compute_chunk(bf)
        out_cp[bf] = pltpu.async_copy(
            out_v[bf],
            out_hbm.at[pl.ds((base + g * _CHUNK) * _D, _CHUNK * _D)],
            sem_out[bf])
    out_cp[(n_chunks - 2) & 1].wait()
    out_cp[(n_chunks - 1) & 1].wait()


def _make_sc_call(n_total):
    n_per_w = n_total // _NW
    mesh = plsc.VectorSubcoreMesh(core_axis_name="c", subcore_axis_name="s")
    return pl.kernel(
        functools.partial(_tiny_model_body, n_per_w),
        out_type=jax.ShapeDtypeStruct((n_total * _D,), jnp.float32),
        mesh=mesh,
        compiler_params=pltpu.CompilerParams(needs_layout_passes=False),
        scratch_types=[
            pltpu.VMEM((_V * _D,), jnp.float32),
            pltpu.VMEM((_CHUNK,), jnp.int32),
            pltpu.VMEM((_CHUNK,), jnp.int32),
            pltpu.VMEM((_CHUNK * _D,), jnp.float32),
            pltpu.VMEM((_CHUNK * _D,), jnp.float32),
            [pltpu.SemaphoreType.DMA, pltpu.SemaphoreType.DMA],
            [pltpu.SemaphoreType.DMA, pltpu.SemaphoreType.DMA],
        ],
    )


def kernel(x, embed_weight, proj_weight, proj_bias):
    b, l = x.shape
    n_total = b * l
    x_flat = x.reshape(-1).astype(jnp.int32)
    out_flat = _make_sc_call(n_total)(
        x_flat,
        embed_weight.reshape(-1).astype(jnp.float32),
        jnp.pad(proj_weight.reshape(-1).astype(jnp.float32), (0, 128 - _D * _D)),
        jnp.pad(proj_bias.astype(jnp.float32), (0, 128 - _D)),
    )
    return out_flat.reshape(b, l, _D)


# speedup vs baseline: 102.0990x; 16.1497x over previous
import functools

import jax
import jax.numpy as jnp
from jax import lax
from jax.experimental import pallas as pl
from jax.experimental.pallas import tpu as pltpu
from jax.experimental.pallas import tpu_sc as plsc

_NC = 2
_NS = 16
_L = 16
_NW = _NC * _NS

_D = 8
_V = 16
_B = 16384
_SEQ = 200
_NLT = _SEQ // _D
_NBT = _B // 128
_BT_PER_W = _NBT // _NW
_UNITS = _BT_PER_W * _NLT
_RING = 4

_XTILE = _D * 128
_OTILE = _D * 128
_OUNIT = _D * _OTILE


def _tiny_model_body(x_hbm, emb_hbm, w_hbm, b_hbm, out_hbm,
                     tab_v, x_v, out_v, sem_x, sem_out):
    wid = lax.axis_index("s") * _NC + lax.axis_index("c")
    bt0 = wid * _BT_PER_W

    lanes = lax.iota(jnp.int32, _L)
    o_pat = lanes & 7
    half = lanes >> 3

    def with_weights(emb_v, w_v, b_v):
        pltpu.sync_copy(emb_hbm, emb_v)
        pltpu.sync_copy(w_hbm, w_v)
        pltpu.sync_copy(b_hbm, b_v)

        def build_tab(j, carry):
            k_pat = 2 * j + half
            acc = plsc.load_gather(b_v, [o_pat])
            for d in range(_D):
                ev = plsc.load_gather(emb_v, [k_pat * _D + d])
                wv = plsc.load_gather(w_v, [o_pat * _D + d])
                acc = acc + ev * wv
            tab_v[pl.ds(j * _L, _L)] = acc
            return carry

        lax.fori_loop(0, _V // 2, build_tab, 0)

    pl.run_scoped(
        with_weights,
        pltpu.VMEM((128,), jnp.float32),
        pltpu.VMEM((128,), jnp.float32),
        pltpu.VMEM((128,), jnp.float32),
    )

    def x_off(u):
        lt = lax.rem(u, _NLT)
        bt = bt0 + lax.div(u, _NLT)
        return (lt * _NBT + bt) * _XTILE

    def start_x(u, slot):
        return pltpu.async_copy(
            x_hbm.at[pl.ds(x_off(lax.min(u, _UNITS - 1)), _XTILE)],
            x_v.at[pl.ds(slot * _XTILE, _XTILE)], sem_x[slot])

    def wait_x(u, slot):
        pltpu.make_async_copy(
            x_hbm.at[pl.ds(x_off(lax.min(u, _UNITS - 1)), _XTILE)],
            x_v.at[pl.ds(slot * _XTILE, _XTILE)], sem_x[slot]).wait()

    def drain_out(slot):
        pltpu.make_async_copy(
            out_hbm.at[pl.ds(0, _OUNIT)],
            out_v.at[pl.ds(slot * _OUNIT, _OUNIT)], sem_out[slot]).wait()

    for slot in range(_RING):
        start_x(jnp.int32(slot), slot)

    def k_body(k, carry):
        for p in range(_RING):
            u = k * _RING + p
            lt = lax.rem(u, _NLT)
            bt = bt0 + lax.div(u, _NLT)
            wait_x(u, p)

            @pl.when(k >= 1)
            def _():
                drain_out(p)

            xbase = p * _XTILE
            obase = p * _OUNIT

            def unit(i):
                ls = i >> 3
                sub = i & 7
                xv = x_v[pl.ds(xbase + ls * 128 + sub * _L, _L)]
                xv8 = xv * _D
                for o in range(_D):
                    val = plsc.load_gather(tab_v, [xv8 + o])
                    out_v[pl.ds(obase + ls * _OTILE + o * 128 + sub * _L,
                                _L)] = val

            plsc.parallel_loop(0, _D * _D, 1, unroll=4)(unit)

            start_x(u + _RING, p)
            for ls in range(_D):
                dst = ((lt * _D + ls) * _NBT + bt) * _OTILE
                pltpu.async_copy(
                    out_v.at[pl.ds(obase + ls * _OTILE, _OTILE)],
                    out_hbm.at[pl.ds(dst, _OTILE)], sem_out[p])
        return carry

    lax.fori_loop(0, _UNITS // _RING, k_body, 0)

    for slot in range(_RING):
        drain_out(slot)
        pltpu.make_async_copy(
            x_hbm.at[pl.ds(0, _XTILE)],
            x_v.at[pl.ds(slot * _XTILE, _XTILE)], sem_x[slot]).wait()


def _make_sc_call():
    mesh = plsc.VectorSubcoreMesh(core_axis_name="c", subcore_axis_name="s")
    return pl.kernel(
        _tiny_model_body,
        out_type=jax.ShapeDtypeStruct((_B * _SEQ * _D,), jnp.float32),
        mesh=mesh,
        compiler_params=pltpu.CompilerParams(needs_layout_passes=False),
        scratch_types=[
            pltpu.VMEM((_V * _D,), jnp.float32),
            pltpu.VMEM((_RING * _XTILE,), jnp.int32),
            pltpu.VMEM((_RING * _OUNIT,), jnp.float32),
            [pltpu.SemaphoreType.DMA] * _RING,
            [pltpu.SemaphoreType.DMA] * _RING,
        ],
    )


def kernel(x, embed_weight, proj_weight, proj_bias):
    b, l = x.shape
    x4 = (x.astype(jnp.int32).T
          .reshape(_NLT, _D, _NBT, 128)
          .transpose(0, 2, 1, 3)
          .reshape(-1))
    out_flat = _make_sc_call()(
        x4,
        embed_weight.reshape(-1).astype(jnp.float32),
        jnp.pad(proj_weight.reshape(-1).astype(jnp.float32), (0, 128 - _D * _D)),
        jnp.pad(proj_bias.astype(jnp.float32), (0, 128 - _D)),
    )
    return (out_flat.reshape(_SEQ, _NBT, _D, 128)
            .transpose(1, 3, 0, 2)
            .reshape(b, l, _D))
